# baseline (speedup 1.0000x reference)
import jax
import jax.numpy as jnp
from jax import lax
from jax.experimental import pallas as pl
from jax.experimental.pallas import tpu as pltpu

_INITIAL_VOCAB = 100000


def _emb_body(idx_smem, emb_in, nemb_in, emb_out, nbuf, tbuf, sem_n, sem_t):
    tok = idx_smem[0]
    exp_row = tok - _INITIAL_VOCAB

    n_in = pltpu.make_async_copy(nemb_in, nbuf, sem_n)
    n_in.start()
    a = (exp_row * 64 // 512) * 512
    e_in = pltpu.make_async_copy(emb_out.at[pl.ds(a, 512)], tbuf, sem_t)
    e_in.start()

    n_in.wait()
    e_in.wait()
    blk = lax.broadcasted_iota(jnp.int32, (512,), 0) // 64
    hit = blk == (exp_row * 64 - a) // 64
    tbuf[...] = jnp.where(hit, nbuf[...], tbuf[...])
    e_out = pltpu.make_async_copy(tbuf, emb_out.at[pl.ds(a, 512)], sem_t)
    e_out.start()
    e_out.wait()


def _cnt_body(idx_smem, usage_in, ctime_in, nemb_in,
              usage_out, ctime_out, row_out,
              nbuf, ubuf, cbuf, sem_n, sem_u, sem_c):
    tok = idx_smem[0]

    n_in = pltpu.make_async_copy(nemb_in, nbuf, sem_n)
    n_in.start()
    au = (tok // 512) * 512
    u_in = pltpu.make_async_copy(usage_out.at[pl.ds(au, 512)], ubuf, sem_u)
    u_in.start()
    c_in = pltpu.make_async_copy(ctime_out.at[pl.ds(au, 512)], cbuf, sem_c)
    c_in.start()

    n_in.wait()
    row_cp = pltpu.make_async_copy(nbuf.at[pl.ds(0, 128)], row_out, sem_n)
    row_cp.start()

    lane = lax.broadcasted_iota(jnp.int32, (512,), 0)
    u_in.wait()
    ubuf[...] = ubuf[...] + (lane == tok - au).astype(jnp.float32)
    u_out = pltpu.make_async_copy(ubuf, usage_out.at[pl.ds(au, 512)], sem_u)
    u_out.start()

    c_in.wait()
    cbuf[...] = jnp.where(lane == tok - au, jnp.float32(jnp.inf), cbuf[...])
    c_out = pltpu.make_async_copy(cbuf, ctime_out.at[pl.ds(au, 512)], sem_c)
    c_out.start()

    row_cp.wait()
    u_out.wait()
    c_out.wait()


def kernel(token_usage, token_creation_time, expanded_embeddings,
           new_embedding, new_token_id):
    idx = jnp.asarray(new_token_id, jnp.int32).reshape(1)
    n_rows, dim = expanded_embeddings.shape
    nemb8 = jnp.tile(new_embedding, 512 // dim)
    emb_flat = expanded_embeddings.reshape(-1)

    expanded = pl.pallas_call(
        _emb_body,
        in_specs=[
            pl.BlockSpec(memory_space=pltpu.SMEM),
            pl.BlockSpec(memory_space=pl.ANY),
            pl.BlockSpec(memory_space=pl.ANY),
        ],
        out_specs=pl.BlockSpec(memory_space=pl.ANY),
        out_shape=jax.ShapeDtypeStruct(emb_flat.shape, jnp.float32),
        input_output_aliases={1: 0},
        scratch_shapes=[
            pltpu.VMEM((512,), jnp.float32),
            pltpu.VMEM((512,), jnp.float32),
            pltpu.SemaphoreType.DMA,
            pltpu.SemaphoreType.DMA,
        ],
    )(idx, emb_flat, nemb8)

    usage, ctime, row = pl.pallas_call(
        _cnt_body,
        in_specs=[
            pl.BlockSpec(memory_space=pltpu.SMEM),
            pl.BlockSpec(memory_space=pl.ANY),
            pl.BlockSpec(memory_space=pl.ANY),
            pl.BlockSpec(memory_space=pl.ANY),
        ],
        out_specs=[
            pl.BlockSpec(memory_space=pl.ANY),
            pl.BlockSpec(memory_space=pl.ANY),
            pl.BlockSpec(memory_space=pl.ANY),
        ],
        out_shape=[
            jax.ShapeDtypeStruct(token_usage.shape, jnp.float32),
            jax.ShapeDtypeStruct(token_creation_time.shape, jnp.float32),
            jax.ShapeDtypeStruct((128,), jnp.float32),
        ],
        input_output_aliases={1: 0, 2: 1},
        scratch_shapes=[
            pltpu.VMEM((512,), jnp.float32),
            pltpu.VMEM((512,), jnp.float32),
            pltpu.VMEM((512,), jnp.float32),
            pltpu.SemaphoreType.DMA,
            pltpu.SemaphoreType.DMA,
            pltpu.SemaphoreType.DMA,
        ],
    )(idx, token_usage, token_creation_time, nemb8)
    return (row[:dim], expanded.reshape(n_rows, dim), usage, ctime)


# speedup vs baseline: 6.9069x; 6.9069x over previous
import jax
import jax.numpy as jnp
from jax import lax
from jax.experimental import pallas as pl
from jax.experimental.pallas import tpu as pltpu

_INITIAL_VOCAB = 100000


def _emb_body(idx_smem, emb_in, nemb_in, emb_out, nbuf, tbuf, sem_n, sem_t):
    tok = idx_smem[0]
    exp_col = tok - _INITIAL_VOCAB

    n_in = pltpu.make_async_copy(nemb_in, nbuf, sem_n)
    n_in.start()
    ac = pl.multiple_of((exp_col // 128) * 128, 128)
    e_in = pltpu.make_async_copy(emb_out.at[:, pl.ds(ac, 128)], tbuf, sem_t)
    e_in.start()

    n_in.wait()
    e_in.wait()
    col = lax.broadcasted_iota(jnp.int32, (64, 128), 1)
    tbuf[...] = jnp.where(col == exp_col - ac, nbuf[...], tbuf[...])
    e_out = pltpu.make_async_copy(tbuf, emb_out.at[:, pl.ds(ac, 128)], sem_t)
    e_out.start()
    e_out.wait()


def _cnt_body(idx_smem, usage_in, ctime_in, nemb_in,
              usage_out, ctime_out, row_out,
              nbuf, ubuf, cbuf, sem_n, sem_u, sem_c):
    tok = idx_smem[0]

    n_in = pltpu.make_async_copy(nemb_in, nbuf, sem_n)
    n_in.start()
    au = (tok // 512) * 512
    u_in = pltpu.make_async_copy(usage_out.at[pl.ds(au, 512)], ubuf, sem_u)
    u_in.start()
    c_in = pltpu.make_async_copy(ctime_out.at[pl.ds(au, 512)], cbuf, sem_c)
    c_in.start()

    n_in.wait()
    row_cp = pltpu.make_async_copy(nbuf, row_out, sem_n)
    row_cp.start()

    lane = lax.broadcasted_iota(jnp.int32, (512,), 0)
    u_in.wait()
    ubuf[...] = ubuf[...] + (lane == tok - au).astype(jnp.float32)
    u_out = pltpu.make_async_copy(ubuf, usage_out.at[pl.ds(au, 512)], sem_u)
    u_out.start()

    c_in.wait()
    cbuf[...] = jnp.where(lane == tok - au, jnp.float32(jnp.inf), cbuf[...])
    c_out = pltpu.make_async_copy(cbuf, ctime_out.at[pl.ds(au, 512)], sem_c)
    c_out.start()

    row_cp.wait()
    u_out.wait()
    c_out.wait()


def kernel(token_usage, token_creation_time, expanded_embeddings,
           new_embedding, new_token_id):
    idx = jnp.asarray(new_token_id, jnp.int32).reshape(1)
    n_rows, dim = expanded_embeddings.shape
    emb_t = expanded_embeddings.T
    nemb_w = jnp.tile(new_embedding.reshape(dim, 1), (1, 128))

    expanded_t = pl.pallas_call(
        _emb_body,
        in_specs=[
            pl.BlockSpec(memory_space=pltpu.SMEM),
            pl.BlockSpec(memory_space=pl.ANY),
            pl.BlockSpec(memory_space=pl.ANY),
        ],
        out_specs=pl.BlockSpec(memory_space=pl.ANY),
        out_shape=jax.ShapeDtypeStruct((dim, n_rows), jnp.float32),
        input_output_aliases={1: 0},
        scratch_shapes=[
            pltpu.VMEM((dim, 128), jnp.float32),
            pltpu.VMEM((dim, 128), jnp.float32),
            pltpu.SemaphoreType.DMA,
            pltpu.SemaphoreType.DMA,
        ],
    )(idx, emb_t, nemb_w)

    usage, ctime, row = pl.pallas_call(
        _cnt_body,
        in_specs=[
            pl.BlockSpec(memory_space=pltpu.SMEM),
            pl.BlockSpec(memory_space=pl.ANY),
            pl.BlockSpec(memory_space=pl.ANY),
            pl.BlockSpec(memory_space=pl.ANY),
        ],
        out_specs=[
            pl.BlockSpec(memory_space=pl.ANY),
            pl.BlockSpec(memory_space=pl.ANY),
            pl.BlockSpec(memory_space=pl.ANY),
        ],
        out_shape=[
            jax.ShapeDtypeStruct(token_usage.shape, jnp.float32),
            jax.ShapeDtypeStruct(token_creation_time.shape, jnp.float32),
            jax.ShapeDtypeStruct((dim, 128), jnp.float32),
        ],
        input_output_aliases={1: 0, 2: 1},
        scratch_shapes=[
            pltpu.VMEM((dim, 128), jnp.float32),
            pltpu.VMEM((512,), jnp.float32),
            pltpu.VMEM((512,), jnp.float32),
            pltpu.SemaphoreType.DMA,
            pltpu.SemaphoreType.DMA,
            pltpu.SemaphoreType.DMA,
        ],
    )(idx, token_usage, token_creation_time, nemb_w)
    return (row[:, 0], expanded_t.T, usage, ctime)


# speedup vs baseline: 6.9708x; 1.0093x over previous
import jax
import jax.numpy as jnp
from jax import lax
from jax.experimental import pallas as pl
from jax.experimental.pallas import tpu as pltpu

_INITIAL_VOCAB = 100000


def _body(idx_smem, emb_in, usage_in, ctime_in, nemb_in,
          emb_out, usage_out, ctime_out, row_out,
          nbuf, tbuf, ubuf, cbuf,
          sem_n, sem_t, sem_u, sem_c):
    tok = idx_smem[0]
    exp_col = tok - _INITIAL_VOCAB

    n_in = pltpu.make_async_copy(nemb_in, nbuf, sem_n)
    n_in.start()
    ac = pl.multiple_of((exp_col // 128) * 128, 128)
    e_in = pltpu.make_async_copy(emb_out.at[:, pl.ds(ac, 128)], tbuf, sem_t)
    e_in.start()
    au = (tok // 512) * 512
    u_in = pltpu.make_async_copy(usage_out.at[pl.ds(au, 512)], ubuf, sem_u)
    u_in.start()
    c_in = pltpu.make_async_copy(ctime_out.at[pl.ds(au, 512)], cbuf, sem_c)
    c_in.start()

    n_in.wait()
    row_cp = pltpu.make_async_copy(nbuf, row_out, sem_n)
    row_cp.start()

    e_in.wait()
    col = lax.broadcasted_iota(jnp.int32, (64, 128), 1)
    tbuf[...] = jnp.where(col == exp_col - ac, nbuf[...], tbuf[...])
    e_out = pltpu.make_async_copy(tbuf, emb_out.at[:, pl.ds(ac, 128)], sem_t)
    e_out.start()

    lane = lax.broadcasted_iota(jnp.int32, (512,), 0)
    u_in.wait()
    ubuf[...] = ubuf[...] + (lane == tok - au).astype(jnp.float32)
    u_out = pltpu.make_async_copy(ubuf, usage_out.at[pl.ds(au, 512)], sem_u)
    u_out.start()

    c_in.wait()
    cbuf[...] = jnp.where(lane == tok - au, jnp.float32(jnp.inf), cbuf[...])
    c_out = pltpu.make_async_copy(cbuf, ctime_out.at[pl.ds(au, 512)], sem_c)
    c_out.start()

    row_cp.wait()
    e_out.wait()
    u_out.wait()
    c_out.wait()


def kernel(token_usage, token_creation_time, expanded_embeddings,
           new_embedding, new_token_id):
    idx = jnp.asarray(new_token_id, jnp.int32).reshape(1)
    n_rows, dim = expanded_embeddings.shape
    emb_t = expanded_embeddings.T
    nemb_w = jnp.tile(new_embedding.reshape(dim, 1), (1, 128))

    expanded_t, usage, ctime, row = pl.pallas_call(
        _body,
        in_specs=[
            pl.BlockSpec(memory_space=pltpu.SMEM),
            pl.BlockSpec(memory_space=pl.ANY),
            pl.BlockSpec(memory_space=pl.ANY),
            pl.BlockSpec(memory_space=pl.ANY),
            pl.BlockSpec(memory_space=pl.ANY),
        ],
        out_specs=[
            pl.BlockSpec(memory_space=pl.ANY),
            pl.BlockSpec(memory_space=pl.ANY),
            pl.BlockSpec(memory_space=pl.ANY),
            pl.BlockSpec(memory_space=pl.ANY),
        ],
        out_shape=[
            jax.ShapeDtypeStruct((dim, n_rows), jnp.float32),
            jax.ShapeDtypeStruct(token_usage.shape, jnp.float32),
            jax.ShapeDtypeStruct(token_creation_time.shape, jnp.float32),
            jax.ShapeDtypeStruct((dim, 128), jnp.float32),
        ],
        input_output_aliases={1: 0, 2: 1, 3: 2},
        scratch_shapes=[
            pltpu.VMEM((dim, 128), jnp.float32),
            pltpu.VMEM((dim, 128), jnp.float32),
            pltpu.VMEM((512,), jnp.float32),
            pltpu.VMEM((512,), jnp.float32),
            pltpu.SemaphoreType.DMA,
            pltpu.SemaphoreType.DMA,
            pltpu.SemaphoreType.DMA,
            pltpu.SemaphoreType.DMA,
        ],
    )(idx, emb_t, token_usage, token_creation_time, nemb_w)
    return (row[:, 0], expanded_t.T, usage, ctime)
